# baseline (speedup 1.0000x reference)
import functools

import jax
import jax.numpy as jnp
from jax import lax
from jax.experimental import pallas as pl
from jax.experimental.pallas import tpu as pltpu
from jax.experimental.pallas import tpu_sc as plsc

_CH = 64


def _sc_gather(table4, idx3, n_rows, d, nbuf=5):
    nw, n_ch, ch = idx3.shape
    per_w = n_ch * ch
    ngrp = n_ch // nbuf
    assert ngrp * nbuf == n_ch

    mesh = plsc.VectorSubcoreMesh(core_axis_name="c", subcore_axis_name="s")
    nc = 2

    @functools.partial(
        pl.kernel,
        mesh=mesh,
        out_type=jax.ShapeDtypeStruct((n_rows, d), jnp.float32),
        scratch_types=[
            pltpu.VMEM((n_ch, ch), jnp.int32),
            pltpu.VMEM((n_ch, ch), jnp.int32),
            pltpu.VMEM((nbuf, ch, 128), jnp.float32),
            pltpu.VMEM((nbuf, ch, d), jnp.float32),
            [pltpu.SemaphoreType.DMA] * nbuf,
            [pltpu.SemaphoreType.DMA] * nbuf,
        ],
    )
    def k(t4_hbm, idx_hbm, out_hbm, idx_v, rows_v, slab_v, pack_v,
          gsems, osems):
        wid = lax.axis_index("s") * nc + lax.axis_index("c")
        base = wid * per_w
        pltpu.sync_copy(idx_hbm.at[wid], idx_v)

        def rowcalc(j, carry):
            for kk in range(ch // 16):
                sl = pl.ds(kk * 16, 16)
                rows_v[j, sl] = lax.shift_right_logical(idx_v[j, sl], 2)
            return carry

        lax.fori_loop(0, n_ch, rowcalc, 0)

        def start_gather(j, b):
            pltpu.async_copy(t4_hbm.at[rows_v.at[j]], slab_v.at[b], gsems[b])

        def wait_gather(j, b):
            pltpu.make_async_copy(
                t4_hbm.at[rows_v.at[j]], slab_v.at[b], gsems[b]
            ).wait()

        def extract(j, b):
            def ebody(g, carry):
                idvec = idx_v[j, pl.ds(g * 16, 16)]
                offs = (idvec & 3) * d
                for l in range(16):
                    i = g * 16 + l
                    off = offs[l]
                    for kk in range(d // 16):
                        pack_v[b, i, pl.ds(kk * 16, 16)] = (
                            slab_v[b, i, pl.ds(off + kk * 16, 16)]
                        )
                return carry

            lax.fori_loop(0, ch // 16, ebody, 0)

        def start_out(j, b):
            pltpu.async_copy(
                pack_v.at[b], out_hbm.at[pl.ds(base + j * ch, ch)], osems[b]
            )

        def wait_out(j, b):
            pltpu.make_async_copy(
                pack_v.at[b], out_hbm.at[pl.ds(base + j * ch, ch)], osems[b]
            ).wait()

        for b in range(nbuf):
            start_gather(b, b)

        def body(p, carry):
            for b in range(nbuf):
                j = p * nbuf + b
                wait_gather(j, b)

                @pl.when(p > 0)
                def _():
                    wait_out(j - nbuf, b)

                extract(j, b)
                start_out(j, b)

                @pl.when(p < ngrp - 1)
                def _():
                    start_gather(j + nbuf, b)

            return carry

        lax.fori_loop(0, ngrp, body, 0)

        jl = (ngrp - 1) * nbuf
        for b in range(nbuf):
            wait_out(jl + b, b)

    return k(table4, idx3)


def _tc_dense(g_flat, mask2, fold, foldT, proj_W, proj_b, ln_g, ln_b,
              att_W1, att_b1, att_w2, B, N, H, GB):
    M = GB * N

    def body(g_ref, m_ref, q_ref, qt_ref, W_ref, pb_ref, lg_ref, lb_ref,
             W1_ref, b1_ref, w2_ref, out_ref, attn_ref):
        x = g_ref[...]
        v = jnp.dot(x, W_ref[...], preferred_element_type=jnp.float32)
        v = jnp.maximum(v + pb_ref[...], 0.0)
        ones_col = jnp.full((H, 1), 1.0 / H, dtype=jnp.float32)
        mu = jnp.dot(v, ones_col, preferred_element_type=jnp.float32)
        m2 = jnp.dot(v * v, ones_col, preferred_element_type=jnp.float32)
        var = jnp.maximum(m2 - mu * mu, 0.0)
        v = (v - mu) * lax.rsqrt(var + 1e-5) * lg_ref[...] + lb_ref[...]
        h = jnp.tanh(
            jnp.dot(v, W1_ref[...], preferred_element_type=jnp.float32)
            + b1_ref[...]
        )
        s = jnp.dot(h, w2_ref[...], preferred_element_type=jnp.float32)
        e = jnp.where(m_ref[...] != 0, jnp.exp(s), 0.0)
        denom_b = jnp.dot(
            q_ref[...], e, preferred_element_type=jnp.float32
        )
        denom_col = jnp.dot(
            qt_ref[...], denom_b, preferred_element_type=jnp.float32
        )
        w_col = e / denom_col
        attn_ref[...] = w_col
        out_ref[...] = jnp.dot(
            q_ref[...], w_col * v, preferred_element_type=jnp.float32
        )

    D = g_flat.shape[1]
    grid = (B // GB,)
    out, attn = pl.pallas_call(
        body,
        grid=grid,
        in_specs=[
            pl.BlockSpec((M, D), lambda i: (i, 0)),
            pl.BlockSpec((M, 1), lambda i: (i, 0)),
            pl.BlockSpec((GB, M), lambda i: (0, 0)),
            pl.BlockSpec((M, GB), lambda i: (0, 0)),
            pl.BlockSpec((D, H), lambda i: (0, 0)),
            pl.BlockSpec((1, H), lambda i: (0, 0)),
            pl.BlockSpec((1, H), lambda i: (0, 0)),
            pl.BlockSpec((1, H), lambda i: (0, 0)),
            pl.BlockSpec((H, H), lambda i: (0, 0)),
            pl.BlockSpec((1, H), lambda i: (0, 0)),
            pl.BlockSpec((H, 1), lambda i: (0, 0)),
        ],
        out_specs=(
            pl.BlockSpec((GB, H), lambda i: (i, 0)),
            pl.BlockSpec((M, 1), lambda i: (i, 0)),
        ),
        out_shape=(
            jax.ShapeDtypeStruct((B, H), jnp.float32),
            jax.ShapeDtypeStruct((B * N, 1), jnp.float32),
        ),
    )(g_flat, mask2, fold, foldT, proj_W, proj_b, ln_g, ln_b, att_W1,
      att_b1, att_w2)
    return out, attn


def kernel(neighbor_user_ids, neighbor_mask, emb_table, proj_W, proj_b,
           ln_g, ln_b, att_W1, att_b1, att_w2, att_b2):
    B, N = neighbor_user_ids.shape
    D = emb_table.shape[1]
    H = att_W1.shape[0]

    info = plsc.get_sparse_core_info()
    nw = info.num_cores * info.num_subcores
    total = B * N
    half = total // 2
    assert half % (nw * _CH) == 0 and (emb_table.shape[0] * D) % 128 == 0
    ids = neighbor_user_ids.reshape(-1).astype(jnp.int32)
    flat = lax.optimization_barrier(emb_table.reshape(-1))
    table4 = flat.reshape(emb_table.shape[0] * D // 128, 128)

    GB = 128
    M = GB * N
    fold = (jax.lax.broadcasted_iota(jnp.int32, (GB, M), 1) // N
            == jax.lax.broadcasted_iota(jnp.int32, (GB, M), 0)
            ).astype(jnp.float32)
    foldT = fold.T
    mask_col = neighbor_mask.reshape(total, 1).astype(jnp.int32)
    Bh = B // 2

    outs = []
    attns = []
    for k2 in range(2):
        idx3 = ids[k2 * half:(k2 + 1) * half].reshape(nw, -1, _CH)
        gathered = _sc_gather(table4, idx3, half, D)
        o, a = _tc_dense(
            gathered, mask_col[k2 * half:(k2 + 1) * half], fold, foldT,
            proj_W, proj_b.reshape(1, H), ln_g.reshape(1, H),
            ln_b.reshape(1, H), att_W1, att_b1.reshape(1, H), att_w2,
            Bh, N, H, GB,
        )
        outs.append(o)
        attns.append(a)

    out = jnp.concatenate(outs, axis=0)
    attn = jnp.concatenate(attns, axis=0)
    return out, attn.reshape(B, N)
